# Initial kernel scaffold; baseline (speedup 1.0000x reference)
import jax
import jax.numpy as jnp
from jax.experimental import pallas as pl


def kernel(vid0, vid1, flows, state):
    raise NotImplementedError("write your pallas kernel here")


# speedup vs baseline: 2.6308x; 2.6308x over previous
import jax
import jax.numpy as jnp
from jax.experimental import pallas as pl
from jax.experimental.pallas import tpu as pltpu

K = 7
PS = 7
WS = 8
STRIDE = 4
H = 256
W = 256
T = 3
C = 32
EXT = H + PS - 1
NQ = H // STRIDE
OFF = WS // 2
CH_SPLIT = 2
CB = C // CH_SPLIT


def _body(a_ref, b_ref, s_ref, dout_ref, iout_ref, dref):
    ch = pl.program_id(1)

    @pl.when(ch == 0)
    def _init():
        dref[...] = jnp.zeros((WS * WS, NQ, NQ), jnp.float32)

    smat = s_ref[...]

    for o in range(WS * WS):
        dh = o // WS - OFF
        dw = o % WS - OFF
        r0 = dh + OFF
        c0 = dw + OFF

        def cbody(c, acc):
            return acc + a_ref[0, c] * b_ref[0, c, r0:r0 + EXT, c0:c0 + EXT]

        g = jax.lax.fori_loop(0, CB, cbody, jnp.zeros((EXT, EXT), jnp.float32))
        gw = jax.lax.dot_general(
            g, smat, (((1,), (0,)), ((), ())),
            precision=jax.lax.Precision.HIGHEST,
            preferred_element_type=jnp.float32)
        t_o = jax.lax.dot_general(
            smat, gw, (((0,), (0,)), ((), ())),
            precision=jax.lax.Precision.HIGHEST,
            preferred_element_type=jnp.float32)
        dref[o] += t_o

    @pl.when(ch == CH_SPLIT - 1)
    def _topk():
        t_idx = pl.program_id(0)
        o_iota = jax.lax.broadcasted_iota(jnp.int32, (WS * WS, NQ, NQ), 0)
        hi = jax.lax.broadcasted_iota(jnp.int32, (NQ, NQ), 0)
        wi = jax.lax.broadcasted_iota(jnp.int32, (NQ, NQ), 1)
        for k in range(K):
            d = dref[...]
            m = jnp.max(d, axis=0)
            sel = jnp.min(jnp.where(d == m[None], o_iota, WS * WS), axis=0)
            dout_ref[0, k] = m
            dh = sel // WS - OFF
            dw = sel % WS - OFF
            iout_ref[0, 0, k] = jnp.zeros((NQ, NQ), jnp.int32) + t_idx
            iout_ref[0, 1, k] = jnp.abs(STRIDE * hi + dh)
            iout_ref[0, 2, k] = jnp.abs(STRIDE * wi + dw)
            dref[...] = jnp.where(o_iota == sel[None], -jnp.inf, d)


def kernel(vid0, vid1, flows, state):
    del flows, state
    a = jnp.pad(vid0[0], ((0, 0), (0, 0), (0, PS - 1), (0, PS - 1)),
                mode="reflect")
    b = jnp.pad(vid1[0], ((0, 0), (0, 0), (OFF, PS - 1 + WS - OFF - 1),
                          (OFF, PS - 1 + WS - OFF - 1)),
                mode="reflect")
    wcol = jnp.arange(EXT)[:, None]
    qcol = jnp.arange(NQ)[None, :] * STRIDE
    smat = ((wcol >= qcol) & (wcol <= qcol + PS - 1)).astype(jnp.float32)

    bext = OFF + PS - 1 + WS - OFF - 1 + H

    d_out, i_out = pl.pallas_call(
        _body,
        grid=(T, CH_SPLIT),
        in_specs=[
            pl.BlockSpec((1, CB, EXT, EXT), lambda t, c: (t, c, 0, 0)),
            pl.BlockSpec((1, CB, bext, bext), lambda t, c: (t, c, 0, 0)),
            pl.BlockSpec((EXT, NQ), lambda t, c: (0, 0)),
        ],
        out_specs=[
            pl.BlockSpec((1, K, NQ, NQ), lambda t, c: (t, 0, 0, 0)),
            pl.BlockSpec((1, 3, K, NQ, NQ), lambda t, c: (t, 0, 0, 0, 0)),
        ],
        out_shape=[
            jax.ShapeDtypeStruct((T, K, NQ, NQ), jnp.float32),
            jax.ShapeDtypeStruct((T, 3, K, NQ, NQ), jnp.int32),
        ],
        scratch_shapes=[pltpu.VMEM((WS * WS, NQ, NQ), jnp.float32)],
        compiler_params=pltpu.CompilerParams(
            dimension_semantics=("arbitrary", "arbitrary")),
    )(a, b, smat)

    nq2 = T * NQ * NQ
    dists = d_out.reshape(T, K, NQ * NQ).transpose(0, 2, 1)
    dists = dists.reshape(1, 1, nq2, K)
    inds = i_out.reshape(T, 3, K, NQ * NQ).transpose(0, 3, 2, 1)
    inds = inds.reshape(1, 1, nq2, K, 3)
    return dists, inds


# speedup vs baseline: 3.8424x; 1.4606x over previous
import jax
import jax.numpy as jnp
from jax.experimental import pallas as pl
from jax.experimental.pallas import tpu as pltpu

K = 7
PS = 7
WS = 8
STRIDE = 4
H = 256
W = 256
T = 3
C = 32
EXT = H + PS - 1
EXTP = 264
BEXTP = 272
NQ = H // STRIDE
OFF = WS // 2
CH_SPLIT = 4
CB = C // CH_SPLIT
SR = 8
NSTRIP = EXTP // SR


def _body(a_ref, b_ref, s_ref, dout_ref, iout_ref, dref, bs_ref, g8_ref):
    ch = pl.program_id(1)

    @pl.when(ch == 0)
    def _init():
        dref[...] = jnp.zeros((WS * WS, NQ, NQ), jnp.float32)

    smat = s_ref[...]

    for dw_i in range(WS):
        bs_ref[...] = b_ref[0, :, :, dw_i:dw_i + EXTP]
        gbuf = dw_i % 2

        def strip_body(i, carry):
            rs = pl.multiple_of(i * SR, SR)
            accs = [jnp.zeros((SR, EXTP), jnp.float32) for _ in range(WS)]
            for c in range(CB):
                av = a_ref[0, c, pl.ds(rs, SR), :]
                bw = bs_ref[c, pl.ds(rs, 2 * SR), :]
                for dhi in range(WS):
                    accs[dhi] = accs[dhi] + av * bw[dhi:dhi + SR]
            for dhi in range(WS):
                g8_ref[gbuf, dhi, pl.ds(rs, SR), :] = accs[dhi]
            return carry

        jax.lax.fori_loop(0, NSTRIP, strip_body, 0)

        for dhi in range(WS):
            g = g8_ref[gbuf, dhi]
            t1 = jax.lax.dot_general(
                smat, g, (((0,), (0,)), ((), ())),
                precision=jax.lax.Precision.HIGHEST,
                preferred_element_type=jnp.float32)
            t_o = jax.lax.dot_general(
                t1, smat, (((1,), (0,)), ((), ())),
                precision=jax.lax.Precision.HIGHEST,
                preferred_element_type=jnp.float32)
            o = dhi * WS + dw_i
            dref[o] += t_o

    @pl.when(ch == CH_SPLIT - 1)
    def _topk():
        t_idx = pl.program_id(0)
        o_iota = jax.lax.broadcasted_iota(jnp.int32, (WS * WS, NQ, NQ), 0)
        hi = jax.lax.broadcasted_iota(jnp.int32, (NQ, NQ), 0)
        wi = jax.lax.broadcasted_iota(jnp.int32, (NQ, NQ), 1)
        for k in range(K):
            d = dref[...]
            m = jnp.max(d, axis=0)
            sel = jnp.min(jnp.where(d == m[None], o_iota, WS * WS), axis=0)
            dout_ref[0, k] = m
            dh = sel // WS - OFF
            dw = sel % WS - OFF
            iout_ref[0, 0, k] = jnp.zeros((NQ, NQ), jnp.int32) + t_idx
            iout_ref[0, 1, k] = jnp.abs(STRIDE * hi + dh)
            iout_ref[0, 2, k] = jnp.abs(STRIDE * wi + dw)
            dref[...] = jnp.where(o_iota == sel[None], -jnp.inf, d)


def _reflect_pad(v, lo, hipad):
    n = v.shape[2]
    parts = []
    if lo:
        parts.append(v[:, :, lo:0:-1, :])
    parts.append(v)
    if hipad:
        parts.append(v[:, :, n - 2:n - 2 - hipad:-1, :])
    v = jnp.concatenate(parts, axis=2)
    n = v.shape[3]
    parts = []
    if lo:
        parts.append(v[:, :, :, lo:0:-1])
    parts.append(v)
    if hipad:
        parts.append(v[:, :, :, n - 2:n - 2 - hipad:-1])
    return jnp.concatenate(parts, axis=3)


def kernel(vid0, vid1, flows, state):
    del flows, state
    a = _reflect_pad(vid0[0], 0, EXTP - H)
    b = _reflect_pad(vid1[0], OFF, BEXTP - OFF - H)
    wcol = jnp.arange(EXTP)[:, None]
    qcol = jnp.arange(NQ)[None, :] * STRIDE
    smat = ((wcol >= qcol) & (wcol <= qcol + PS - 1)).astype(jnp.float32)

    d_out, i_out = pl.pallas_call(
        _body,
        grid=(T, CH_SPLIT),
        in_specs=[
            pl.BlockSpec((1, CB, EXTP, EXTP), lambda t, c: (t, c, 0, 0)),
            pl.BlockSpec((1, CB, BEXTP, BEXTP), lambda t, c: (t, c, 0, 0)),
            pl.BlockSpec((EXTP, NQ), lambda t, c: (0, 0)),
        ],
        out_specs=[
            pl.BlockSpec((1, K, NQ, NQ), lambda t, c: (t, 0, 0, 0)),
            pl.BlockSpec((1, 3, K, NQ, NQ), lambda t, c: (t, 0, 0, 0, 0)),
        ],
        out_shape=[
            jax.ShapeDtypeStruct((T, K, NQ, NQ), jnp.float32),
            jax.ShapeDtypeStruct((T, 3, K, NQ, NQ), jnp.int32),
        ],
        scratch_shapes=[
            pltpu.VMEM((WS * WS, NQ, NQ), jnp.float32),
            pltpu.VMEM((CB, BEXTP, EXTP), jnp.float32),
            pltpu.VMEM((2, WS, EXTP, EXTP), jnp.float32),
        ],
        compiler_params=pltpu.CompilerParams(
            dimension_semantics=("arbitrary", "arbitrary")),
    )(a, b, smat)

    nq2 = T * NQ * NQ
    dists = d_out.reshape(T, K, NQ * NQ).transpose(0, 2, 1)
    dists = dists.reshape(1, 1, nq2, K)
    inds = i_out.reshape(T, 3, K, NQ * NQ).transpose(0, 3, 2, 1)
    inds = inds.reshape(1, 1, nq2, K, 3)
    return dists, inds


# speedup vs baseline: 4.7149x; 1.2271x over previous
import jax
import jax.numpy as jnp
from jax.experimental import pallas as pl
from jax.experimental.pallas import tpu as pltpu

K = 7
PS = 7
WS = 8
STRIDE = 4
H = 256
W = 256
T = 3
C = 32
EXT = H + PS - 1
EXTP = 264
BEXTP = 272
NQ = H // STRIDE
OFF = WS // 2
CH_SPLIT = 2
CB = C // CH_SPLIT
SR = 8
NSTRIP = EXTP // SR


def _body(a_ref, b_ref, s_ref, dout_ref, iout_ref, dref, bs_ref, g8_ref):
    ch = pl.program_id(1)

    @pl.when(ch == 0)
    def _init():
        dref[...] = jnp.zeros((WS * WS, NQ, NQ), jnp.float32)

    smat = s_ref[...]

    for dw_i in range(WS):
        bs_ref[...] = b_ref[0, :, :, dw_i:dw_i + EXTP]
        gbuf = dw_i % 2

        def strip_body(i, carry):
            rs = pl.multiple_of(i * SR, SR)
            accs = [jnp.zeros((SR, EXTP), jnp.float32) for _ in range(WS)]
            for c in range(CB):
                av = a_ref[0, c, pl.ds(rs, SR), :]
                bw = bs_ref[c, pl.ds(rs, 2 * SR), :]
                for dhi in range(WS):
                    accs[dhi] = accs[dhi] + av * bw[dhi:dhi + SR]
            for dhi in range(WS):
                g8_ref[gbuf, dhi, pl.ds(rs, SR), :] = accs[dhi]
            return carry

        jax.lax.fori_loop(0, NSTRIP, strip_body, 0)

        for dhi in range(WS):
            g = g8_ref[gbuf, dhi]
            t1 = jax.lax.dot_general(
                smat, g, (((0,), (0,)), ((), ())),
                precision=jax.lax.Precision.HIGHEST,
                preferred_element_type=jnp.float32)
            t_o = jax.lax.dot_general(
                t1, smat, (((1,), (0,)), ((), ())),
                precision=jax.lax.Precision.HIGHEST,
                preferred_element_type=jnp.float32)
            o = dhi * WS + dw_i
            dref[o] += t_o

    @pl.when(ch == CH_SPLIT - 1)
    def _topk():
        t_idx = pl.program_id(0)
        o_iota = jax.lax.broadcasted_iota(jnp.int32, (WS * WS, NQ, NQ), 0)
        hi = jax.lax.broadcasted_iota(jnp.int32, (NQ, NQ), 0)
        wi = jax.lax.broadcasted_iota(jnp.int32, (NQ, NQ), 1)
        for k in range(K):
            d = dref[...]
            m = jnp.max(d, axis=0)
            sel = jnp.min(jnp.where(d == m[None], o_iota, WS * WS), axis=0)
            dout_ref[0, k] = m
            dh = sel // WS - OFF
            dw = sel % WS - OFF
            iout_ref[0, 0, k] = jnp.zeros((NQ, NQ), jnp.int32) + t_idx
            iout_ref[0, 1, k] = jnp.abs(STRIDE * hi + dh)
            iout_ref[0, 2, k] = jnp.abs(STRIDE * wi + dw)
            dref[...] = jnp.where(o_iota == sel[None], -jnp.inf, d)


def _reflect_pad(v, lo, hipad):
    n = v.shape[2]
    parts = []
    if lo:
        parts.append(v[:, :, lo:0:-1, :])
    parts.append(v)
    if hipad:
        parts.append(v[:, :, n - 2:n - 2 - hipad:-1, :])
    v = jnp.concatenate(parts, axis=2)
    n = v.shape[3]
    parts = []
    if lo:
        parts.append(v[:, :, :, lo:0:-1])
    parts.append(v)
    if hipad:
        parts.append(v[:, :, :, n - 2:n - 2 - hipad:-1])
    return jnp.concatenate(parts, axis=3)


def kernel(vid0, vid1, flows, state):
    del flows, state
    a = _reflect_pad(vid0[0], 0, EXTP - H)
    b = _reflect_pad(vid1[0], OFF, BEXTP - OFF - H)
    wcol = jnp.arange(EXTP)[:, None]
    qcol = jnp.arange(NQ)[None, :] * STRIDE
    smat = ((wcol >= qcol) & (wcol <= qcol + PS - 1)).astype(jnp.float32)

    d_out, i_out = pl.pallas_call(
        _body,
        grid=(T, CH_SPLIT),
        in_specs=[
            pl.BlockSpec((1, CB, EXTP, EXTP), lambda t, c: (t, c, 0, 0)),
            pl.BlockSpec((1, CB, BEXTP, BEXTP), lambda t, c: (t, c, 0, 0)),
            pl.BlockSpec((EXTP, NQ), lambda t, c: (0, 0)),
        ],
        out_specs=[
            pl.BlockSpec((1, K, NQ, NQ), lambda t, c: (t, 0, 0, 0)),
            pl.BlockSpec((1, 3, K, NQ, NQ), lambda t, c: (t, 0, 0, 0, 0)),
        ],
        out_shape=[
            jax.ShapeDtypeStruct((T, K, NQ, NQ), jnp.float32),
            jax.ShapeDtypeStruct((T, 3, K, NQ, NQ), jnp.int32),
        ],
        scratch_shapes=[
            pltpu.VMEM((WS * WS, NQ, NQ), jnp.float32),
            pltpu.VMEM((CB, BEXTP, EXTP), jnp.float32),
            pltpu.VMEM((2, WS, EXTP, EXTP), jnp.float32),
        ],
        compiler_params=pltpu.CompilerParams(
            dimension_semantics=("arbitrary", "arbitrary")),
    )(a, b, smat)

    nq2 = T * NQ * NQ
    dists = d_out.reshape(T, K, NQ * NQ).transpose(0, 2, 1)
    dists = dists.reshape(1, 1, nq2, K)
    inds = i_out.reshape(T, 3, K, NQ * NQ).transpose(0, 3, 2, 1)
    inds = inds.reshape(1, 1, nq2, K, 3)
    return dists, inds


# speedup vs baseline: 7.3821x; 1.5657x over previous
import jax
import jax.numpy as jnp
from jax.experimental import pallas as pl
from jax.experimental.pallas import tpu as pltpu

K = 7
PS = 7
WS = 8
STRIDE = 4
H = 256
W = 256
T = 3
C = 32
EXT = H + PS - 1
EXTP = 264
BEXTP = 272
NQ = H // STRIDE
OFF = WS // 2
CH_SPLIT = 2
CB = C // CH_SPLIT
SR = 8
NSTRIP = EXTP // SR


def _body(a_ref, b_ref, s_ref, dout_ref, iout_ref, dref, ap_ref, bp_ref,
          bs_ref, g8_ref):
    ch = pl.program_id(1)

    @pl.when(ch == 0)
    def _init():
        dref[...] = jnp.zeros((WS * WS, NQ, NQ), jnp.float32)

    smat = s_ref[...]

    ap_ref[:, 0:H, 0:H] = a_ref[0, 0]
    for j in range(EXTP - H):
        ap_ref[:, H + j, 0:H] = a_ref[0, 0, :, H - 2 - j, :]
    for j in range(EXTP - H):
        ap_ref[:, :, H + j:H + j + 1] = ap_ref[:, :, H - 2 - j:H - 1 - j]

    bp_ref[:, OFF:OFF + H, OFF:OFF + H] = b_ref[0, 0]
    for j in range(1, OFF + 1):
        bp_ref[:, OFF - j, OFF:OFF + H] = b_ref[0, 0, :, j, :]
    for j in range(BEXTP - OFF - H):
        bp_ref[:, OFF + H + j, OFF:OFF + H] = b_ref[0, 0, :, H - 2 - j, :]
    for j in range(1, OFF + 1):
        bp_ref[:, :, OFF - j:OFF - j + 1] = bp_ref[:, :, OFF + j:OFF + j + 1]
    for j in range(BEXTP - OFF - H):
        bp_ref[:, :, OFF + H + j:OFF + H + j + 1] = (
            bp_ref[:, :, OFF + H - 2 - j:OFF + H - 1 - j])

    for dw_i in range(WS):
        bs_ref[...] = bp_ref[:, :, dw_i:dw_i + EXTP]
        gbuf = dw_i % 2

        def strip_body(i, carry):
            rs = pl.multiple_of(i * SR, SR)
            accs = [jnp.zeros((SR, EXTP), jnp.float32) for _ in range(WS)]
            for c in range(CB):
                av = ap_ref[c, pl.ds(rs, SR), :]
                bw = bs_ref[c, pl.ds(rs, 2 * SR), :]
                for dhi in range(WS):
                    accs[dhi] = accs[dhi] + av * bw[dhi:dhi + SR]
            for dhi in range(WS):
                g8_ref[gbuf, dhi, pl.ds(rs, SR), :] = accs[dhi]
            return carry

        jax.lax.fori_loop(0, NSTRIP, strip_body, 0)

        for dhi in range(WS):
            g = g8_ref[gbuf, dhi]
            t1 = jax.lax.dot_general(
                smat, g, (((0,), (0,)), ((), ())),
                precision=jax.lax.Precision.HIGHEST,
                preferred_element_type=jnp.float32)
            t_o = jax.lax.dot_general(
                t1, smat, (((1,), (0,)), ((), ())),
                precision=jax.lax.Precision.HIGHEST,
                preferred_element_type=jnp.float32)
            o = dhi * WS + dw_i
            dref[o] += t_o

    @pl.when(ch == CH_SPLIT - 1)
    def _topk():
        t_idx = pl.program_id(0)
        o_iota = jax.lax.broadcasted_iota(jnp.int32, (WS * WS, NQ, NQ), 0)
        hi = jax.lax.broadcasted_iota(jnp.int32, (NQ, NQ), 0)
        wi = jax.lax.broadcasted_iota(jnp.int32, (NQ, NQ), 1)
        for k in range(K):
            d = dref[...]
            m = jnp.max(d, axis=0)
            sel = jnp.min(jnp.where(d == m[None], o_iota, WS * WS), axis=0)
            dout_ref[0, k] = m
            dh = sel // WS - OFF
            dw = sel % WS - OFF
            iout_ref[0, 0, k] = jnp.zeros((NQ, NQ), jnp.int32) + t_idx
            iout_ref[0, 1, k] = jnp.abs(STRIDE * hi + dh)
            iout_ref[0, 2, k] = jnp.abs(STRIDE * wi + dw)
            dref[...] = jnp.where(o_iota == sel[None], -jnp.inf, d)


def _reflect_pad(v, lo, hipad):
    n = v.shape[2]
    parts = []
    if lo:
        parts.append(v[:, :, lo:0:-1, :])
    parts.append(v)
    if hipad:
        parts.append(v[:, :, n - 2:n - 2 - hipad:-1, :])
    v = jnp.concatenate(parts, axis=2)
    n = v.shape[3]
    parts = []
    if lo:
        parts.append(v[:, :, :, lo:0:-1])
    parts.append(v)
    if hipad:
        parts.append(v[:, :, :, n - 2:n - 2 - hipad:-1])
    return jnp.concatenate(parts, axis=3)


def kernel(vid0, vid1, flows, state):
    del flows, state
    wcol = jnp.arange(EXTP)[:, None]
    qcol = jnp.arange(NQ)[None, :] * STRIDE
    smat = ((wcol >= qcol) & (wcol <= qcol + PS - 1)).astype(jnp.float32)

    d_out, i_out = pl.pallas_call(
        _body,
        grid=(T, CH_SPLIT),
        in_specs=[
            pl.BlockSpec((1, 1, CB, H, H), lambda t, c: (0, t, c, 0, 0)),
            pl.BlockSpec((1, 1, CB, H, H), lambda t, c: (0, t, c, 0, 0)),
            pl.BlockSpec((EXTP, NQ), lambda t, c: (0, 0)),
        ],
        out_specs=[
            pl.BlockSpec((1, K, NQ, NQ), lambda t, c: (t, 0, 0, 0)),
            pl.BlockSpec((1, 3, K, NQ, NQ), lambda t, c: (t, 0, 0, 0, 0)),
        ],
        out_shape=[
            jax.ShapeDtypeStruct((T, K, NQ, NQ), jnp.float32),
            jax.ShapeDtypeStruct((T, 3, K, NQ, NQ), jnp.int32),
        ],
        scratch_shapes=[
            pltpu.VMEM((WS * WS, NQ, NQ), jnp.float32),
            pltpu.VMEM((CB, EXTP, EXTP), jnp.float32),
            pltpu.VMEM((CB, BEXTP, BEXTP), jnp.float32),
            pltpu.VMEM((CB, BEXTP, EXTP), jnp.float32),
            pltpu.VMEM((2, WS, EXTP, EXTP), jnp.float32),
        ],
        compiler_params=pltpu.CompilerParams(
            dimension_semantics=("arbitrary", "arbitrary")),
    )(vid0, vid1, smat)

    nq2 = T * NQ * NQ
    dists = d_out.reshape(T, K, NQ * NQ).transpose(0, 2, 1)
    dists = dists.reshape(1, 1, nq2, K)
    inds = i_out.reshape(T, 3, K, NQ * NQ).transpose(0, 3, 2, 1)
    inds = inds.reshape(1, 1, nq2, K, 3)
    return dists, inds


# speedup vs baseline: 7.3975x; 1.0021x over previous
import jax
import jax.numpy as jnp
from jax.experimental import pallas as pl
from jax.experimental.pallas import tpu as pltpu

K = 7
PS = 7
WS = 8
STRIDE = 4
H = 256
W = 256
T = 3
C = 32
EXT = H + PS - 1
EXTP = 264
BEXTP = 272
NQ = H // STRIDE
OFF = WS // 2
CH_SPLIT = 2
CB = C // CH_SPLIT
SR = 8
NSTRIP = EXTP // SR


def _body(a_ref, b_ref, s_ref, dout_ref, iout_ref, dref, ap_ref, bp_ref,
          bs_ref, g8_ref):
    ch = pl.program_id(1)

    @pl.when(ch == 0)
    def _init():
        dref[...] = jnp.zeros((WS * WS, NQ, NQ), jnp.float32)

    smat = s_ref[...]

    ap_ref[:, 0:H, 0:H] = a_ref[0, 0]
    for j in range(EXTP - H):
        ap_ref[:, H + j, 0:H] = a_ref[0, 0, :, H - 2 - j, :]
    for j in range(EXTP - H):
        ap_ref[:, :, H + j:H + j + 1] = ap_ref[:, :, H - 2 - j:H - 1 - j]

    bp_ref[:, OFF:OFF + H, OFF:OFF + H] = b_ref[0, 0]
    for j in range(1, OFF + 1):
        bp_ref[:, OFF - j, OFF:OFF + H] = b_ref[0, 0, :, j, :]
    for j in range(BEXTP - OFF - H):
        bp_ref[:, OFF + H + j, OFF:OFF + H] = b_ref[0, 0, :, H - 2 - j, :]
    for j in range(1, OFF + 1):
        bp_ref[:, :, OFF - j:OFF - j + 1] = bp_ref[:, :, OFF + j:OFF + j + 1]
    for j in range(BEXTP - OFF - H):
        bp_ref[:, :, OFF + H + j:OFF + H + j + 1] = (
            bp_ref[:, :, OFF + H - 2 - j:OFF + H - 1 - j])

    for dw_i in range(WS):
        bs_ref[...] = bp_ref[:, :, dw_i:dw_i + EXTP]
        gbuf = dw_i % 2

        def strip_body(i, carry):
            rs = pl.multiple_of(i * SR, SR)
            accs = [jnp.zeros((SR, EXTP), jnp.float32) for _ in range(WS)]
            for c in range(CB):
                av = ap_ref[c, pl.ds(rs, SR), :]
                bw = bs_ref[c, pl.ds(rs, 2 * SR), :]
                for dhi in range(WS):
                    accs[dhi] = accs[dhi] + av * bw[dhi:dhi + SR]
            for dhi in range(WS):
                g8_ref[gbuf, dhi, pl.ds(rs, SR), :] = accs[dhi]
            return carry

        jax.lax.fori_loop(0, NSTRIP, strip_body, 0, unroll=3)

        for dhi in range(WS):
            g = g8_ref[gbuf, dhi]
            t1 = jax.lax.dot_general(
                smat, g, (((0,), (0,)), ((), ())),
                precision=jax.lax.Precision.HIGHEST,
                preferred_element_type=jnp.float32)
            t_o = jax.lax.dot_general(
                t1, smat, (((1,), (0,)), ((), ())),
                precision=jax.lax.Precision.HIGHEST,
                preferred_element_type=jnp.float32)
            o = dhi * WS + dw_i
            dref[o] += t_o

    @pl.when(ch == CH_SPLIT - 1)
    def _topk():
        t_idx = pl.program_id(0)
        o_iota = jax.lax.broadcasted_iota(jnp.int32, (WS * WS, NQ, NQ), 0)
        hi = jax.lax.broadcasted_iota(jnp.int32, (NQ, NQ), 0)
        wi = jax.lax.broadcasted_iota(jnp.int32, (NQ, NQ), 1)
        for k in range(K):
            d = dref[...]
            m = jnp.max(d, axis=0)
            sel = jnp.min(jnp.where(d == m[None], o_iota, WS * WS), axis=0)
            dout_ref[0, k] = m
            dh = sel // WS - OFF
            dw = sel % WS - OFF
            iout_ref[0, 0, k] = jnp.zeros((NQ, NQ), jnp.int32) + t_idx
            iout_ref[0, 1, k] = jnp.abs(STRIDE * hi + dh)
            iout_ref[0, 2, k] = jnp.abs(STRIDE * wi + dw)
            dref[...] = jnp.where(o_iota == sel[None], -jnp.inf, d)


def _reflect_pad(v, lo, hipad):
    n = v.shape[2]
    parts = []
    if lo:
        parts.append(v[:, :, lo:0:-1, :])
    parts.append(v)
    if hipad:
        parts.append(v[:, :, n - 2:n - 2 - hipad:-1, :])
    v = jnp.concatenate(parts, axis=2)
    n = v.shape[3]
    parts = []
    if lo:
        parts.append(v[:, :, :, lo:0:-1])
    parts.append(v)
    if hipad:
        parts.append(v[:, :, :, n - 2:n - 2 - hipad:-1])
    return jnp.concatenate(parts, axis=3)


def kernel(vid0, vid1, flows, state):
    del flows, state
    wcol = jnp.arange(EXTP)[:, None]
    qcol = jnp.arange(NQ)[None, :] * STRIDE
    smat = ((wcol >= qcol) & (wcol <= qcol + PS - 1)).astype(jnp.float32)

    d_out, i_out = pl.pallas_call(
        _body,
        grid=(T, CH_SPLIT),
        in_specs=[
            pl.BlockSpec((1, 1, CB, H, H), lambda t, c: (0, t, c, 0, 0)),
            pl.BlockSpec((1, 1, CB, H, H), lambda t, c: (0, t, c, 0, 0)),
            pl.BlockSpec((EXTP, NQ), lambda t, c: (0, 0)),
        ],
        out_specs=[
            pl.BlockSpec((1, K, NQ, NQ), lambda t, c: (t, 0, 0, 0)),
            pl.BlockSpec((1, 3, K, NQ, NQ), lambda t, c: (t, 0, 0, 0, 0)),
        ],
        out_shape=[
            jax.ShapeDtypeStruct((T, K, NQ, NQ), jnp.float32),
            jax.ShapeDtypeStruct((T, 3, K, NQ, NQ), jnp.int32),
        ],
        scratch_shapes=[
            pltpu.VMEM((WS * WS, NQ, NQ), jnp.float32),
            pltpu.VMEM((CB, EXTP, EXTP), jnp.float32),
            pltpu.VMEM((CB, BEXTP, BEXTP), jnp.float32),
            pltpu.VMEM((CB, BEXTP, EXTP), jnp.float32),
            pltpu.VMEM((2, WS, EXTP, EXTP), jnp.float32),
        ],
        compiler_params=pltpu.CompilerParams(
            dimension_semantics=("arbitrary", "arbitrary")),
    )(vid0, vid1, smat)

    nq2 = T * NQ * NQ
    dists = d_out.reshape(T, K, NQ * NQ).transpose(0, 2, 1)
    dists = dists.reshape(1, 1, nq2, K)
    inds = i_out.reshape(T, 3, K, NQ * NQ).transpose(0, 3, 2, 1)
    inds = inds.reshape(1, 1, nq2, K, 3)
    return dists, inds
